# baseline (speedup 1.0000x reference)
import jax
import jax.numpy as jnp
from jax import lax
from jax.experimental import pallas as pl
from jax.experimental.pallas import tpu as pltpu
from jax.experimental.pallas import tpu_sc as plsc

N = 50000
E = 800000
G = 64
NROW = 50176
GARBAGE = 50000
NTILE = 32
EPT = 25088
EPAD = NTILE * EPT
B = 128
NB = EPT // B
STRIPE = NROW // 16
ZROWS = 98
SLICES = 8
RBLK = 512
NBLK = NROW // RBLK
NSLOT = 4
NO = NB // NSLOT
DEGK = 8

_SELU_L = 1.0507009873554805
_SELU_A = 1.6732632423543772


def _selu(x):
  return _SELU_L * jnp.where(x > 0, x, _SELU_A * (jnp.exp(x) - 1.0))


def _sc_mesh():
  return plsc.VectorSubcoreMesh(core_axis_name="c", subcore_axis_name="s")


_SC_PARAMS = pltpu.CompilerParams(use_tc_tiling_on_sc=False)


def _zero_fill(zbuf, width):
  @pl.loop(0, ZROWS)
  def _(r):
    for k in range(width // 16):
      zbuf[r, pl.ds(k * 16, 16)] = jnp.zeros((16,), jnp.float32)


def _zero_acc(acc, zbuf, t0):
  for j in range(STRIPE // ZROWS):
    pltpu.sync_copy(zbuf, acc.at[pl.ds(t0 + j * ZROWS, ZROWS)])


def _deg_body(dst_hbm, out_hbm, acc, dstidx, ones, zbuf, sem):
  c = lax.axis_index("c")
  s = lax.axis_index("s")
  w = s * 2 + c
  t0 = s * STRIPE
  _zero_fill(zbuf, 16)

  @pl.loop(0, B)
  def _(r):
    ones[r, :] = jnp.ones((16,), jnp.float32)

  _zero_acc(acc, zbuf, t0)
  plsc.subcore_barrier()

  @pl.loop(0, NB)
  def _(i):
    pltpu.sync_copy(dst_hbm.at[w, i], dstidx)
    pltpu.sync_copy(ones, acc.at[dstidx], add=True)

  plsc.subcore_barrier()
  pltpu.sync_copy(acc.at[pl.ds(t0, STRIPE)], out_hbm.at[c, pl.ds(t0, STRIPE)])


def _sc_deg(dst_p):
  return pl.kernel(
      _deg_body,
      out_type=jax.ShapeDtypeStruct((2, NROW, 16), jnp.float32),
      mesh=_sc_mesh(),
      compiler_params=_SC_PARAMS,
      scratch_types=[
          pltpu.VMEM_SHARED((NROW, 16), jnp.float32),
          pltpu.VMEM((B,), jnp.int32),
          pltpu.VMEM((B, 16), jnp.float32),
          pltpu.VMEM((ZROWS, 16), jnp.float32),
          pltpu.SemaphoreType.DMA,
      ],
  )(dst_p)


def _make_agg_body(n_slices, width):
  def body(src_hbm, dst_hbm, *refs):
    gts = refs[:n_slices]
    out_hbm = refs[n_slices]
    acc, zbuf = refs[n_slices + 1:n_slices + 3]
    base = n_slices + 3
    srcb = refs[base:base + NSLOT]
    dstb = refs[base + NSLOT:base + 2 * NSLOT]
    rows = refs[base + 2 * NSLOT:base + 3 * NSLOT]
    isems = refs[base + 3 * NSLOT:base + 4 * NSLOT]
    gsems = refs[base + 4 * NSLOT:base + 5 * NSLOT]
    ssems = refs[base + 5 * NSLOT:base + 6 * NSLOT]
    c = lax.axis_index("c")
    s = lax.axis_index("s")
    w = s * 2 + c
    t0 = s * STRIPE
    _zero_fill(zbuf, width)

    def sidx(g, j):
      return pltpu.make_async_copy(src_hbm.at[w, pl.ds(g * B, B)], srcb[j],
                                   isems[j])

    def didx(g, j):
      return pltpu.make_async_copy(dst_hbm.at[w, g], dstb[j], isems[j])

    for sl in range(n_slices):
      table = gts[sl]

      def gth(j, table=table):
        return pltpu.make_async_copy(table.at[srcb[j]], rows[j], gsems[j])

      def sct(j):
        return pltpu.make_async_copy(rows[j], acc.at[dstb[j]], ssems[j])

      _zero_acc(acc, zbuf, t0)
      plsc.subcore_barrier()
      for j in range(NSLOT):
        sidx(j, j).start()
        didx(j, j).start()
      for j in range(NSLOT):
        sidx(j, j).wait()
        didx(j, j).wait()
        gth(j).start()

      @pl.loop(0, NO)
      def _(o):
        g0 = o * NSLOT
        for j in range(NSLOT):
          g = g0 + j
          gth(j).wait()

          @pl.when(g < NB - NSLOT)
          def _():
            sidx(g + NSLOT, j).start()

          sct(j).start(add=True)
        for j in range(NSLOT):
          g = g0 + j
          sct(j).wait()

          @pl.when(g < NB - NSLOT)
          def _():
            didx(g + NSLOT, j).start()
            sidx(g + NSLOT, j).wait()
            didx(g + NSLOT, j).wait()
            gth(j).start()

      plsc.subcore_barrier()
      pltpu.sync_copy(acc.at[pl.ds(t0, STRIPE)],
                      out_hbm.at[2 * sl + c, pl.ds(t0, STRIPE)])
      plsc.subcore_barrier()

  return body


def _sc_agg(src_p, dst_p, tables, width):
  n_slices = len(tables)
  return pl.kernel(
      _make_agg_body(n_slices, width),
      out_type=jax.ShapeDtypeStruct((2 * n_slices, NROW, width),
                                    jnp.float32),
      mesh=_sc_mesh(),
      compiler_params=_SC_PARAMS,
      scratch_types=[
          pltpu.VMEM_SHARED((NROW, width), jnp.float32),
          pltpu.VMEM((ZROWS, width), jnp.float32),
      ] + [pltpu.VMEM((B,), jnp.int32) for _ in range(NSLOT)]
        + [pltpu.VMEM((B,), jnp.int32) for _ in range(NSLOT)]
        + [pltpu.VMEM((B, width), jnp.float32) for _ in range(NSLOT)]
        + [pltpu.SemaphoreType.DMA for _ in range(3 * NSLOT)],
  )(src_p, dst_p, *tables)


def _prep_body(dp_ref, x_ref, dinv_ref, xt_ref):
  deg = dp_ref[0] + dp_ref[1] + 1.0
  dinv = lax.rsqrt(deg)
  dinv_ref[...] = dinv
  xt_ref[...] = x_ref[...] * dinv


def _tc_prep(deg_part, x16):
  return pl.pallas_call(
      _prep_body,
      grid=(NBLK,),
      in_specs=[
          pl.BlockSpec((2, RBLK, 16), lambda i: (0, i, 0)),
          pl.BlockSpec((RBLK, 16), lambda i: (i, 0)),
      ],
      out_specs=[
          pl.BlockSpec((RBLK, 16), lambda i: (i, 0)),
          pl.BlockSpec((RBLK, 16), lambda i: (i, 0)),
      ],
      out_shape=[
          jax.ShapeDtypeStruct((NROW, 16), jnp.float32),
          jax.ShapeDtypeStruct((NROW, 16), jnp.float32),
      ],
  )(deg_part, x16)


def _mid_body(ap_ref, xt_ref, dinv_ref, w1_ref, b1_ref, w2_ref, *out_refs):
  dinv = dinv_ref[...]
  s1 = dinv * (ap_ref[0] + ap_ref[1] + xt_ref[...])
  h1 = _selu(jnp.dot(s1, w1_ref[...],
                     preferred_element_type=jnp.float32) + b1_ref[...])
  dcol = dinv[:, 0:1]
  for sl in range(SLICES):
    g = jnp.dot(h1, w2_ref[sl], preferred_element_type=jnp.float32)
    out_refs[sl][...] = g * dcol


def _tc_mid(agg1_part, xt, dinv16, W1p, b1, W2r):
  return pl.pallas_call(
      _mid_body,
      grid=(NBLK,),
      in_specs=[
          pl.BlockSpec((2, RBLK, 16), lambda i: (0, i, 0)),
          pl.BlockSpec((RBLK, 16), lambda i: (i, 0)),
          pl.BlockSpec((RBLK, 16), lambda i: (i, 0)),
          pl.BlockSpec((16, 256), lambda i: (0, 0)),
          pl.BlockSpec((1, 256), lambda i: (0, 0)),
          pl.BlockSpec((SLICES, 256, 32), lambda i: (0, 0, 0)),
      ],
      out_specs=[pl.BlockSpec((RBLK, 32), lambda i: (i, 0))
                 for _ in range(SLICES)],
      out_shape=[jax.ShapeDtypeStruct((NROW, 32), jnp.float32)
                 for _ in range(SLICES)],
  )(agg1_part, xt, dinv16, W1p, b1, W2r)


def _final_body(ap_ref, *refs):
  gt_refs = refs[:SLICES]
  (dinv_ref, bt_ref, b2_ref, fc1w_ref, fc1b_ref, fc2w_ref, fc2b_ref,
   out_ref, sums_ref, counts_ref) = refs[SLICES:]
  i = pl.program_id(0)

  @pl.when(i == 0)
  def _():
    sums_ref[...] = jnp.zeros_like(sums_ref)
    counts_ref[...] = jnp.zeros_like(counts_ref)

  dcol = dinv_ref[...][:, 0:1]
  b = bt_ref[0]
  gids = lax.broadcasted_iota(jnp.int32, (G, RBLK), 0)
  p = (gids == b).astype(jnp.float32)
  counts_ref[...] += jnp.dot(p, jnp.ones((RBLK, 128), jnp.float32),
                             preferred_element_type=jnp.float32)
  for sl in range(SLICES):
    agg = ap_ref[2 * sl] + ap_ref[2 * sl + 1] + gt_refs[sl][...]
    h2 = _selu(dcol * agg + b2_ref[0, sl])
    sums_ref[sl] += jnp.dot(p, h2, preferred_element_type=jnp.float32)

  @pl.when(i == NBLK - 1)
  def _():
    cnt = jnp.maximum(counts_ref[...][:, 0:1], 1.0)
    z1 = jnp.zeros((G, 128), jnp.float32)
    for sl in range(SLICES):
      pooled = _selu(sums_ref[sl] / cnt)
      z1 = z1 + jnp.dot(pooled, fc1w_ref[sl],
                        preferred_element_type=jnp.float32)
    a1 = _selu(z1 + fc1b_ref[...])
    z2 = jnp.dot(a1, fc2w_ref[...],
                 preferred_element_type=jnp.float32) + fc2b_ref[...]
    m = jnp.max(z2, axis=1, keepdims=True)
    lse = m + jnp.log(jnp.sum(jnp.exp(z2 - m), axis=1, keepdims=True))
    out_ref[...] = z2 - lse


def _tc_final(agg2_part, gts, dinv16, batch3, b2r, fc1r, fc1b, fc2w, fc2b):
  return pl.pallas_call(
      _final_body,
      grid=(NBLK,),
      in_specs=[
          pl.BlockSpec((2 * SLICES, RBLK, 32), lambda i: (0, i, 0)),
      ] + [pl.BlockSpec((RBLK, 32), lambda i: (i, 0))
           for _ in range(SLICES)] + [
          pl.BlockSpec((RBLK, 16), lambda i: (i, 0)),
          pl.BlockSpec((1, 1, RBLK), lambda i: (i, 0, 0)),
          pl.BlockSpec((1, SLICES, 32), lambda i: (0, 0, 0)),
          pl.BlockSpec((SLICES, 32, 128), lambda i: (0, 0, 0)),
          pl.BlockSpec((1, 128), lambda i: (0, 0)),
          pl.BlockSpec((128, 2), lambda i: (0, 0)),
          pl.BlockSpec((1, 2), lambda i: (0, 0)),
      ],
      out_specs=pl.BlockSpec((G, 2), lambda i: (0, 0)),
      out_shape=jax.ShapeDtypeStruct((G, 2), jnp.float32),
      scratch_shapes=[
          pltpu.VMEM((SLICES, G, 32), jnp.float32),
          pltpu.VMEM((G, 128), jnp.float32),
      ],
  )(agg2_part, *gts, dinv16, batch3, b2r, fc1r, fc1b, fc2w, fc2b)


@jax.jit
def kernel(x, edge_index, batch, W1, b1, W2, b2, fc1_w, fc1_b, fc2_w, fc2_b):
  src = edge_index[0]
  dst = edge_index[1]
  pad = EPAD - E
  src_p = jnp.concatenate([src, jnp.zeros((pad,), jnp.int32)])
  dst_p = jnp.concatenate([dst, jnp.full((pad,), GARBAGE, jnp.int32)])
  src_p = src_p.reshape(NTILE, EPT)
  dst_p = dst_p.reshape(NTILE, NB, B)

  x16 = jnp.zeros((NROW, 16), jnp.float32).at[:N, :4].set(x)
  batch3 = jnp.full((NROW,), -1, jnp.int32).at[:N].set(batch)
  batch3 = batch3.reshape(NBLK, 1, RBLK)

  W1p = jnp.zeros((16, 2 * 128), jnp.float32).at[:4].set(W1)
  b1r = b1.reshape(1, 256)
  W2r = jnp.transpose(W2.reshape(256, SLICES, 32), (1, 0, 2))
  b2r = b2.reshape(1, SLICES, 32)
  fc1r = fc1_w.reshape(SLICES, 32, 128)
  fc1b = fc1_b.reshape(1, 128)
  fc2b = fc2_b.reshape(1, 2)

  deg_part = _sc_deg(dst_p)
  dinv16, xt = _tc_prep(deg_part, x16)
  agg1_part = _sc_agg(src_p, dst_p, [xt], 16)
  gts = _tc_mid(agg1_part, xt, dinv16, W1p, b1r, W2r)
  agg2_part = _sc_agg(src_p, dst_p, list(gts), 32)
  return _tc_final(agg2_part, gts, dinv16, batch3, b2r, fc1r, fc1b,
                   fc2_w, fc2b)


# speedup vs baseline: 1.2679x; 1.2679x over previous
import jax
import jax.numpy as jnp
from jax import lax
from jax.experimental import pallas as pl
from jax.experimental.pallas import tpu as pltpu
from jax.experimental.pallas import tpu_sc as plsc

N = 50000
E = 800000
G = 64
NROW = 50176
GARBAGE = 50000
NTILE = 32
EPT = 25088
EPAD = NTILE * EPT
B = 128
NB = EPT // B
STRIPE = NROW // 16
ZROWS = 98
SLICES = 8
RBLK = 512
NBLK = NROW // RBLK
NSLOT = 4
NO = NB // NSLOT
DEGK = 8

_SELU_L = 1.0507009873554805
_SELU_A = 1.6732632423543772


def _selu(x):
  return _SELU_L * jnp.where(x > 0, x, _SELU_A * (jnp.exp(x) - 1.0))


def _sc_mesh():
  return plsc.VectorSubcoreMesh(core_axis_name="c", subcore_axis_name="s")


_SC_PARAMS = pltpu.CompilerParams(use_tc_tiling_on_sc=False)


def _zero_fill(zbuf, width):
  @pl.loop(0, ZROWS)
  def _(r):
    for k in range(width // 16):
      zbuf[r, pl.ds(k * 16, 16)] = jnp.zeros((16,), jnp.float32)


def _zero_acc(acc, zbuf, t0):
  for j in range(STRIPE // ZROWS):
    pltpu.sync_copy(zbuf, acc.at[pl.ds(t0 + j * ZROWS, ZROWS)])


def _deg_body(dst_hbm, out_hbm, acc, dstidx, ones, zbuf, sem):
  c = lax.axis_index("c")
  s = lax.axis_index("s")
  w = s * 2 + c
  t0 = s * STRIPE
  _zero_fill(zbuf, 16)

  @pl.loop(0, B)
  def _(r):
    ones[r, :] = jnp.ones((16,), jnp.float32)

  _zero_acc(acc, zbuf, t0)
  plsc.subcore_barrier()

  @pl.loop(0, NB)
  def _(i):
    pltpu.sync_copy(dst_hbm.at[w, i], dstidx)
    pltpu.sync_copy(ones, acc.at[dstidx], add=True)

  plsc.subcore_barrier()
  pltpu.sync_copy(acc.at[pl.ds(t0, STRIPE)], out_hbm.at[c, pl.ds(t0, STRIPE)])


def _sc_deg(dst_p):
  return pl.kernel(
      _deg_body,
      out_type=jax.ShapeDtypeStruct((2, NROW, 16), jnp.float32),
      mesh=_sc_mesh(),
      compiler_params=_SC_PARAMS,
      scratch_types=[
          pltpu.VMEM_SHARED((NROW, 16), jnp.float32),
          pltpu.VMEM((B,), jnp.int32),
          pltpu.VMEM((B, 16), jnp.float32),
          pltpu.VMEM((ZROWS, 16), jnp.float32),
          pltpu.SemaphoreType.DMA,
      ],
  )(dst_p)


def _make_agg_body(n_slices, width, bsz, nslot, split):
  shares = 16 if split else NTILE
  eps = EPAD // shares
  nb = eps // bsz
  no = nb // nslot

  def body(src_hbm, dst_hbm, *refs):
    gts = refs[:n_slices]
    out_hbm = refs[n_slices]
    acc, zbuf = refs[n_slices + 1:n_slices + 3]
    base = n_slices + 3
    srcb = refs[base:base + nslot]
    dstb = refs[base + nslot:base + 2 * nslot]
    rows = refs[base + 2 * nslot:base + 3 * nslot]
    isems = refs[base + 3 * nslot:base + 4 * nslot]
    gsems = refs[base + 4 * nslot:base + 5 * nslot]
    ssems = refs[base + 5 * nslot:base + 6 * nslot]
    c = lax.axis_index("c")
    s = lax.axis_index("s")
    share = s if split else s * 2 + c
    t0 = s * STRIPE
    _zero_fill(zbuf, width)

    def sidx(g, j):
      return pltpu.make_async_copy(src_hbm.at[share, pl.ds(g * bsz, bsz)],
                                   srcb[j], isems[j])

    def didx(g, j):
      return pltpu.make_async_copy(dst_hbm.at[share, g], dstb[j], isems[j])

    for sl in range(n_slices):
      table = gts[sl]

      def gth(j, table=table):
        return pltpu.make_async_copy(table.at[srcb[j]], rows[j], gsems[j])

      def sct(j):
        return pltpu.make_async_copy(rows[j], acc.at[dstb[j]], ssems[j])

      def one_pass():
        _zero_acc(acc, zbuf, t0)
        plsc.subcore_barrier()
        for j in range(nslot):
          sidx(j, j).start()
          didx(j, j).start()
        for j in range(nslot):
          sidx(j, j).wait()
          didx(j, j).wait()
          gth(j).start()

        @pl.loop(0, no)
        def _(o):
          g0 = o * nslot
          for j in range(nslot):
            g = g0 + j
            gth(j).wait()

            @pl.when(g < nb - nslot)
            def _():
              sidx(g + nslot, j).start()

            sct(j).start(add=True)
          for j in range(nslot):
            g = g0 + j
            sct(j).wait()

            @pl.when(g < nb - nslot)
            def _():
              didx(g + nslot, j).start()
              sidx(g + nslot, j).wait()
              didx(g + nslot, j).wait()
              gth(j).start()

        plsc.subcore_barrier()
        out_k = sl if split else 2 * sl + c
        pltpu.sync_copy(acc.at[pl.ds(t0, STRIPE)],
                        out_hbm.at[out_k, pl.ds(t0, STRIPE)])
        plsc.subcore_barrier()

      if split:
        @pl.when(c == sl // (n_slices // 2))
        def _():
          one_pass()
      else:
        one_pass()

  return body


def _sc_agg(src_p, dst_p, tables, width, bsz, nslot, split):
  n_slices = len(tables)
  shares = 16 if split else NTILE
  eps = EPAD // shares
  n_out = n_slices if split else 2 * n_slices
  return pl.kernel(
      _make_agg_body(n_slices, width, bsz, nslot, split),
      out_type=jax.ShapeDtypeStruct((n_out, NROW, width), jnp.float32),
      mesh=_sc_mesh(),
      compiler_params=_SC_PARAMS,
      scratch_types=[
          pltpu.VMEM_SHARED((NROW, width), jnp.float32),
          pltpu.VMEM((ZROWS, width), jnp.float32),
      ] + [pltpu.VMEM((bsz,), jnp.int32) for _ in range(nslot)]
        + [pltpu.VMEM((bsz,), jnp.int32) for _ in range(nslot)]
        + [pltpu.VMEM((bsz, width), jnp.float32) for _ in range(nslot)]
        + [pltpu.SemaphoreType.DMA for _ in range(3 * nslot)],
  )(src_p.reshape(shares, eps),
    dst_p.reshape(shares, eps // bsz, bsz), *tables)


def _prep_body(dp_ref, x_ref, dinv_ref, xt_ref):
  deg = dp_ref[0] + dp_ref[1] + 1.0
  dinv = lax.rsqrt(deg)
  dinv_ref[...] = dinv
  xt_ref[...] = x_ref[...] * dinv


def _tc_prep(deg_part, x16):
  return pl.pallas_call(
      _prep_body,
      grid=(NBLK,),
      in_specs=[
          pl.BlockSpec((2, RBLK, 16), lambda i: (0, i, 0)),
          pl.BlockSpec((RBLK, 16), lambda i: (i, 0)),
      ],
      out_specs=[
          pl.BlockSpec((RBLK, 16), lambda i: (i, 0)),
          pl.BlockSpec((RBLK, 16), lambda i: (i, 0)),
      ],
      out_shape=[
          jax.ShapeDtypeStruct((NROW, 16), jnp.float32),
          jax.ShapeDtypeStruct((NROW, 16), jnp.float32),
      ],
  )(deg_part, x16)


def _mid_body(ap_ref, xt_ref, dinv_ref, w1_ref, b1_ref, w2_ref, *out_refs):
  dinv = dinv_ref[...]
  s1 = dinv * (ap_ref[0] + ap_ref[1] + xt_ref[...])
  h1 = _selu(jnp.dot(s1, w1_ref[...],
                     preferred_element_type=jnp.float32) + b1_ref[...])
  dcol = dinv[:, 0:1]
  for sl in range(SLICES):
    g = jnp.dot(h1, w2_ref[sl], preferred_element_type=jnp.float32)
    out_refs[sl][...] = g * dcol


def _tc_mid(agg1_part, xt, dinv16, W1p, b1, W2r):
  return pl.pallas_call(
      _mid_body,
      grid=(NBLK,),
      in_specs=[
          pl.BlockSpec((2, RBLK, 16), lambda i: (0, i, 0)),
          pl.BlockSpec((RBLK, 16), lambda i: (i, 0)),
          pl.BlockSpec((RBLK, 16), lambda i: (i, 0)),
          pl.BlockSpec((16, 256), lambda i: (0, 0)),
          pl.BlockSpec((1, 256), lambda i: (0, 0)),
          pl.BlockSpec((SLICES, 256, 32), lambda i: (0, 0, 0)),
      ],
      out_specs=[pl.BlockSpec((RBLK, 32), lambda i: (i, 0))
                 for _ in range(SLICES)],
      out_shape=[jax.ShapeDtypeStruct((NROW, 32), jnp.float32)
                 for _ in range(SLICES)],
  )(agg1_part, xt, dinv16, W1p, b1, W2r)


def _final_body(ap_ref, *refs):
  gt_refs = refs[:SLICES]
  (dinv_ref, bt_ref, b2_ref, fc1w_ref, fc1b_ref, fc2w_ref, fc2b_ref,
   out_ref, sums_ref, counts_ref) = refs[SLICES:]
  i = pl.program_id(0)

  @pl.when(i == 0)
  def _():
    sums_ref[...] = jnp.zeros_like(sums_ref)
    counts_ref[...] = jnp.zeros_like(counts_ref)

  dcol = dinv_ref[...][:, 0:1]
  b = bt_ref[0]
  gids = lax.broadcasted_iota(jnp.int32, (G, RBLK), 0)
  p = (gids == b).astype(jnp.float32)
  counts_ref[...] += jnp.dot(p, jnp.ones((RBLK, 128), jnp.float32),
                             preferred_element_type=jnp.float32)
  for sl in range(SLICES):
    agg = ap_ref[sl] + gt_refs[sl][...]
    h2 = _selu(dcol * agg + b2_ref[0, sl])
    sums_ref[sl] += jnp.dot(p, h2, preferred_element_type=jnp.float32)

  @pl.when(i == NBLK - 1)
  def _():
    cnt = jnp.maximum(counts_ref[...][:, 0:1], 1.0)
    z1 = jnp.zeros((G, 128), jnp.float32)
    for sl in range(SLICES):
      pooled = _selu(sums_ref[sl] / cnt)
      z1 = z1 + jnp.dot(pooled, fc1w_ref[sl],
                        preferred_element_type=jnp.float32)
    a1 = _selu(z1 + fc1b_ref[...])
    z2 = jnp.dot(a1, fc2w_ref[...],
                 preferred_element_type=jnp.float32) + fc2b_ref[...]
    m = jnp.max(z2, axis=1, keepdims=True)
    lse = m + jnp.log(jnp.sum(jnp.exp(z2 - m), axis=1, keepdims=True))
    out_ref[...] = z2 - lse


def _tc_final(agg2_part, gts, dinv16, batch3, b2r, fc1r, fc1b, fc2w, fc2b):
  return pl.pallas_call(
      _final_body,
      grid=(NBLK,),
      in_specs=[
          pl.BlockSpec((SLICES, RBLK, 32), lambda i: (0, i, 0)),
      ] + [pl.BlockSpec((RBLK, 32), lambda i: (i, 0))
           for _ in range(SLICES)] + [
          pl.BlockSpec((RBLK, 16), lambda i: (i, 0)),
          pl.BlockSpec((1, 1, RBLK), lambda i: (i, 0, 0)),
          pl.BlockSpec((1, SLICES, 32), lambda i: (0, 0, 0)),
          pl.BlockSpec((SLICES, 32, 128), lambda i: (0, 0, 0)),
          pl.BlockSpec((1, 128), lambda i: (0, 0)),
          pl.BlockSpec((128, 2), lambda i: (0, 0)),
          pl.BlockSpec((1, 2), lambda i: (0, 0)),
      ],
      out_specs=pl.BlockSpec((G, 2), lambda i: (0, 0)),
      out_shape=jax.ShapeDtypeStruct((G, 2), jnp.float32),
      scratch_shapes=[
          pltpu.VMEM((SLICES, G, 32), jnp.float32),
          pltpu.VMEM((G, 128), jnp.float32),
      ],
  )(agg2_part, *gts, dinv16, batch3, b2r, fc1r, fc1b, fc2w, fc2b)


@jax.jit
def kernel(x, edge_index, batch, W1, b1, W2, b2, fc1_w, fc1_b, fc2_w, fc2_b):
  src = edge_index[0]
  dst = edge_index[1]
  pad = EPAD - E
  src_p = jnp.concatenate([src, jnp.zeros((pad,), jnp.int32)])
  dst_p = jnp.concatenate([dst, jnp.full((pad,), GARBAGE, jnp.int32)])

  x16 = jnp.zeros((NROW, 16), jnp.float32).at[:N, :4].set(x)
  batch3 = jnp.full((NROW,), -1, jnp.int32).at[:N].set(batch)
  batch3 = batch3.reshape(NBLK, 1, RBLK)

  W1p = jnp.zeros((16, 2 * 128), jnp.float32).at[:4].set(W1)
  b1r = b1.reshape(1, 256)
  W2r = jnp.transpose(W2.reshape(256, SLICES, 32), (1, 0, 2))
  b2r = b2.reshape(1, SLICES, 32)
  fc1r = fc1_w.reshape(SLICES, 32, 128)
  fc1b = fc1_b.reshape(1, 128)
  fc2b = fc2_b.reshape(1, 2)

  deg_part = _sc_deg(dst_p.reshape(NTILE, NB, B))
  dinv16, xt = _tc_prep(deg_part, x16)
  agg1_part = _sc_agg(src_p, dst_p, [xt], 16, 128, 4, False)
  gts = _tc_mid(agg1_part, xt, dinv16, W1p, b1r, W2r)
  agg2_part = _sc_agg(src_p, dst_p, list(gts), 32, 256, 2, True)
  return _tc_final(agg2_part, gts, dinv16, batch3, b2r, fc1r, fc1b,
                   fc2_w, fc2b)


# speedup vs baseline: 1.3250x; 1.0450x over previous
import jax
import jax.numpy as jnp
from jax import lax
from jax.experimental import pallas as pl
from jax.experimental.pallas import tpu as pltpu
from jax.experimental.pallas import tpu_sc as plsc

N = 50000
E = 800000
G = 64
NROW = 50176
GARBAGE = 50000
NTILE = 32
EPT = 25088
EPAD = NTILE * EPT
B = 128
NB = EPT // B
STRIPE = NROW // 16
ZROWS = 98
SLICES = 8
RBLK = 1024
NBLK = NROW // RBLK
NSLOT = 4
NO = NB // NSLOT
DEGK = 8

_SELU_L = 1.0507009873554805
_SELU_A = 1.6732632423543772


def _selu(x):
  return _SELU_L * jnp.where(x > 0, x, _SELU_A * (jnp.exp(x) - 1.0))


def _sc_mesh():
  return plsc.VectorSubcoreMesh(core_axis_name="c", subcore_axis_name="s")


_SC_PARAMS = pltpu.CompilerParams(use_tc_tiling_on_sc=False)


def _zero_fill(zbuf, width):
  @pl.loop(0, ZROWS)
  def _(r):
    for k in range(width // 16):
      zbuf[r, pl.ds(k * 16, 16)] = jnp.zeros((16,), jnp.float32)


def _zero_acc(acc, zbuf, t0):
  for j in range(STRIPE // ZROWS):
    pltpu.sync_copy(zbuf, acc.at[pl.ds(t0 + j * ZROWS, ZROWS)])


def _deg_body(dst_hbm, out_hbm, acc, dstidx, ones, zbuf, sem):
  c = lax.axis_index("c")
  s = lax.axis_index("s")
  w = s * 2 + c
  t0 = s * STRIPE
  _zero_fill(zbuf, 16)

  @pl.loop(0, B)
  def _(r):
    ones[r, :] = jnp.ones((16,), jnp.float32)

  _zero_acc(acc, zbuf, t0)
  plsc.subcore_barrier()

  @pl.loop(0, NB)
  def _(i):
    pltpu.sync_copy(dst_hbm.at[w, i], dstidx)
    pltpu.sync_copy(ones, acc.at[dstidx], add=True)

  plsc.subcore_barrier()
  pltpu.sync_copy(acc.at[pl.ds(t0, STRIPE)], out_hbm.at[c, pl.ds(t0, STRIPE)])


def _sc_deg(dst_p):
  return pl.kernel(
      _deg_body,
      out_type=jax.ShapeDtypeStruct((2, NROW, 16), jnp.float32),
      mesh=_sc_mesh(),
      compiler_params=_SC_PARAMS,
      scratch_types=[
          pltpu.VMEM_SHARED((NROW, 16), jnp.float32),
          pltpu.VMEM((B,), jnp.int32),
          pltpu.VMEM((B, 16), jnp.float32),
          pltpu.VMEM((ZROWS, 16), jnp.float32),
          pltpu.SemaphoreType.DMA,
      ],
  )(dst_p)


def _make_agg_body(n_slices, width, bsz, nslot, split):
  shares = 16 if split else NTILE
  eps = EPAD // shares
  nb = eps // bsz
  no = nb // nslot

  def body(src_hbm, dst_hbm, *refs):
    gts = refs[:n_slices]
    out_hbm = refs[n_slices]
    acc, zbuf = refs[n_slices + 1:n_slices + 3]
    base = n_slices + 3
    srcb = refs[base:base + nslot]
    dstb = refs[base + nslot:base + 2 * nslot]
    rows = refs[base + 2 * nslot:base + 3 * nslot]
    isems = refs[base + 3 * nslot:base + 4 * nslot]
    gsems = refs[base + 4 * nslot:base + 5 * nslot]
    ssems = refs[base + 5 * nslot:base + 6 * nslot]
    c = lax.axis_index("c")
    s = lax.axis_index("s")
    share = s if split else s * 2 + c
    t0 = s * STRIPE
    _zero_fill(zbuf, width)

    def sidx(g, j):
      return pltpu.make_async_copy(src_hbm.at[share, pl.ds(g * bsz, bsz)],
                                   srcb[j], isems[j])

    def didx(g, j):
      return pltpu.make_async_copy(dst_hbm.at[share, g], dstb[j], isems[j])

    for sl in range(n_slices):
      table = gts[sl]

      def gth(j, table=table):
        return pltpu.make_async_copy(table.at[srcb[j]], rows[j], gsems[j])

      def sct(j):
        return pltpu.make_async_copy(rows[j], acc.at[dstb[j]], ssems[j])

      def one_pass():
        _zero_acc(acc, zbuf, t0)
        plsc.subcore_barrier()
        for j in range(nslot):
          sidx(j, j).start()
          didx(j, j).start()
        for j in range(nslot):
          sidx(j, j).wait()
          didx(j, j).wait()
          gth(j).start()

        @pl.loop(0, no)
        def _(o):
          g0 = o * nslot
          for j in range(nslot):
            g = g0 + j
            gth(j).wait()

            @pl.when(g < nb - nslot)
            def _():
              sidx(g + nslot, j).start()

            sct(j).start(add=True)
          for j in range(nslot):
            g = g0 + j
            sct(j).wait()

            @pl.when(g < nb - nslot)
            def _():
              didx(g + nslot, j).start()
              sidx(g + nslot, j).wait()
              didx(g + nslot, j).wait()
              gth(j).start()

        plsc.subcore_barrier()
        out_k = sl if split else 2 * sl + c
        pltpu.sync_copy(acc.at[pl.ds(t0, STRIPE)],
                        out_hbm.at[out_k, pl.ds(t0, STRIPE)])
        plsc.subcore_barrier()

      if split:
        @pl.when(c == sl // (n_slices // 2))
        def _():
          one_pass()
      else:
        one_pass()

  return body


def _sc_agg(src_p, dst_p, tables, width, bsz, nslot, split):
  n_slices = len(tables)
  shares = 16 if split else NTILE
  eps = EPAD // shares
  n_out = n_slices if split else 2 * n_slices
  return pl.kernel(
      _make_agg_body(n_slices, width, bsz, nslot, split),
      out_type=jax.ShapeDtypeStruct((n_out, NROW, width), jnp.float32),
      mesh=_sc_mesh(),
      compiler_params=_SC_PARAMS,
      scratch_types=[
          pltpu.VMEM_SHARED((NROW, width), jnp.float32),
          pltpu.VMEM((ZROWS, width), jnp.float32),
      ] + [pltpu.VMEM((bsz,), jnp.int32) for _ in range(nslot)]
        + [pltpu.VMEM((bsz,), jnp.int32) for _ in range(nslot)]
        + [pltpu.VMEM((bsz, width), jnp.float32) for _ in range(nslot)]
        + [pltpu.SemaphoreType.DMA for _ in range(3 * nslot)],
  )(src_p.reshape(shares, eps),
    dst_p.reshape(shares, eps // bsz, bsz), *tables)


def _prep_body(dp_ref, x_ref, dinv_ref, xt_ref):
  deg = dp_ref[0] + dp_ref[1] + 1.0
  dinv = lax.rsqrt(deg)
  dinv_ref[...] = dinv
  xt_ref[...] = x_ref[...] * dinv


def _tc_prep(deg_part, x16):
  return pl.pallas_call(
      _prep_body,
      grid=(NBLK,),
      in_specs=[
          pl.BlockSpec((2, RBLK, 16), lambda i: (0, i, 0)),
          pl.BlockSpec((RBLK, 16), lambda i: (i, 0)),
      ],
      out_specs=[
          pl.BlockSpec((RBLK, 16), lambda i: (i, 0)),
          pl.BlockSpec((RBLK, 16), lambda i: (i, 0)),
      ],
      out_shape=[
          jax.ShapeDtypeStruct((NROW, 16), jnp.float32),
          jax.ShapeDtypeStruct((NROW, 16), jnp.float32),
      ],
  )(deg_part, x16)


def _mid_body(ap_ref, xt_ref, dinv_ref, w1_ref, b1_ref, w2_ref, *out_refs):
  dinv = dinv_ref[...]
  s1 = dinv * (ap_ref[0] + ap_ref[1] + xt_ref[...])
  h1 = _selu(jnp.dot(s1, w1_ref[...],
                     preferred_element_type=jnp.float32) + b1_ref[...])
  dcol = dinv[:, 0:1]
  for sl in range(SLICES):
    g = jnp.dot(h1, w2_ref[sl], preferred_element_type=jnp.float32)
    out_refs[sl][...] = g * dcol


def _tc_mid(agg1_part, xt, dinv16, W1p, b1, W2r):
  return pl.pallas_call(
      _mid_body,
      grid=(NBLK,),
      in_specs=[
          pl.BlockSpec((2, RBLK, 16), lambda i: (0, i, 0)),
          pl.BlockSpec((RBLK, 16), lambda i: (i, 0)),
          pl.BlockSpec((RBLK, 16), lambda i: (i, 0)),
          pl.BlockSpec((16, 256), lambda i: (0, 0)),
          pl.BlockSpec((1, 256), lambda i: (0, 0)),
          pl.BlockSpec((SLICES, 256, 32), lambda i: (0, 0, 0)),
      ],
      out_specs=[pl.BlockSpec((RBLK, 32), lambda i: (i, 0))
                 for _ in range(SLICES)],
      out_shape=[jax.ShapeDtypeStruct((NROW, 32), jnp.float32)
                 for _ in range(SLICES)],
  )(agg1_part, xt, dinv16, W1p, b1, W2r)


def _final_body(ap_ref, *refs):
  gt_refs = refs[:SLICES]
  (dinv_ref, bt_ref, b2_ref, fc1w_ref, fc1b_ref, fc2w_ref, fc2b_ref,
   out_ref, sums_ref, counts_ref) = refs[SLICES:]
  i = pl.program_id(0)

  @pl.when(i == 0)
  def _():
    sums_ref[...] = jnp.zeros_like(sums_ref)
    counts_ref[...] = jnp.zeros_like(counts_ref)

  dcol = dinv_ref[...][:, 0:1]
  b = bt_ref[0]
  gids = lax.broadcasted_iota(jnp.int32, (G, RBLK), 0)
  p = (gids == b).astype(jnp.float32)
  counts_ref[...] += jnp.dot(p, jnp.ones((RBLK, 128), jnp.float32),
                             preferred_element_type=jnp.float32)
  for sl in range(SLICES):
    agg = ap_ref[sl] + gt_refs[sl][...]
    h2 = _selu(dcol * agg + b2_ref[0, sl])
    sums_ref[sl] += jnp.dot(p, h2, preferred_element_type=jnp.float32)

  @pl.when(i == NBLK - 1)
  def _():
    cnt = jnp.maximum(counts_ref[...][:, 0:1], 1.0)
    z1 = jnp.zeros((G, 128), jnp.float32)
    for sl in range(SLICES):
      pooled = _selu(sums_ref[sl] / cnt)
      z1 = z1 + jnp.dot(pooled, fc1w_ref[sl],
                        preferred_element_type=jnp.float32)
    a1 = _selu(z1 + fc1b_ref[...])
    z2 = jnp.dot(a1, fc2w_ref[...],
                 preferred_element_type=jnp.float32) + fc2b_ref[...]
    m = jnp.max(z2, axis=1, keepdims=True)
    lse = m + jnp.log(jnp.sum(jnp.exp(z2 - m), axis=1, keepdims=True))
    out_ref[...] = z2 - lse


def _tc_final(agg2_part, gts, dinv16, batch3, b2r, fc1r, fc1b, fc2w, fc2b):
  return pl.pallas_call(
      _final_body,
      grid=(NBLK,),
      in_specs=[
          pl.BlockSpec((SLICES, RBLK, 32), lambda i: (0, i, 0)),
      ] + [pl.BlockSpec((RBLK, 32), lambda i: (i, 0))
           for _ in range(SLICES)] + [
          pl.BlockSpec((RBLK, 16), lambda i: (i, 0)),
          pl.BlockSpec((1, 1, RBLK), lambda i: (i, 0, 0)),
          pl.BlockSpec((1, SLICES, 32), lambda i: (0, 0, 0)),
          pl.BlockSpec((SLICES, 32, 128), lambda i: (0, 0, 0)),
          pl.BlockSpec((1, 128), lambda i: (0, 0)),
          pl.BlockSpec((128, 2), lambda i: (0, 0)),
          pl.BlockSpec((1, 2), lambda i: (0, 0)),
      ],
      out_specs=pl.BlockSpec((G, 2), lambda i: (0, 0)),
      out_shape=jax.ShapeDtypeStruct((G, 2), jnp.float32),
      scratch_shapes=[
          pltpu.VMEM((SLICES, G, 32), jnp.float32),
          pltpu.VMEM((G, 128), jnp.float32),
      ],
  )(agg2_part, *gts, dinv16, batch3, b2r, fc1r, fc1b, fc2w, fc2b)


@jax.jit
def kernel(x, edge_index, batch, W1, b1, W2, b2, fc1_w, fc1_b, fc2_w, fc2_b):
  src = edge_index[0]
  dst = edge_index[1]
  pad = EPAD - E
  src_p = jnp.concatenate([src, jnp.zeros((pad,), jnp.int32)])
  dst_p = jnp.concatenate([dst, jnp.full((pad,), GARBAGE, jnp.int32)])

  x16 = jnp.zeros((NROW, 16), jnp.float32).at[:N, :4].set(x)
  batch3 = jnp.full((NROW,), -1, jnp.int32).at[:N].set(batch)
  batch3 = batch3.reshape(NBLK, 1, RBLK)

  W1p = jnp.zeros((16, 2 * 128), jnp.float32).at[:4].set(W1)
  b1r = b1.reshape(1, 256)
  W2r = jnp.transpose(W2.reshape(256, SLICES, 32), (1, 0, 2))
  b2r = b2.reshape(1, SLICES, 32)
  fc1r = fc1_w.reshape(SLICES, 32, 128)
  fc1b = fc1_b.reshape(1, 128)
  fc2b = fc2_b.reshape(1, 2)

  deg_part = _sc_deg(dst_p.reshape(NTILE, NB, B))
  dinv16, xt = _tc_prep(deg_part, x16)
  agg1_part = _sc_agg(src_p, dst_p, [xt], 16, 128, 4, False)
  gts = _tc_mid(agg1_part, xt, dinv16, W1p, b1r, W2r)
  agg2_part = _sc_agg(src_p, dst_p, list(gts), 32, 256, 2, True)
  return _tc_final(agg2_part, gts, dinv16, batch3, b2r, fc1r, fc1b,
                   fc2_w, fc2b)


# speedup vs baseline: 1.4004x; 1.0569x over previous
import jax
import jax.numpy as jnp
from jax import lax
from jax.experimental import pallas as pl
from jax.experimental.pallas import tpu as pltpu
from jax.experimental.pallas import tpu_sc as plsc

N = 50000
E = 800000
G = 64
NROW = 50176
GARBAGE = 50000
NTILE = 32
EPT = 25088
EPAD = NTILE * EPT
B = 128
NB = EPT // B
STRIPE = NROW // 16
ZROWS = 98
SLICES = 8
RBLK = 1024
NBLK = NROW // RBLK
NSLOT = 4
NO = NB // NSLOT
DEGK = 8

_SELU_L = 1.0507009873554805
_SELU_A = 1.6732632423543772


def _selu(x):
  return _SELU_L * jnp.where(x > 0, x, _SELU_A * (jnp.exp(x) - 1.0))


def _sc_mesh():
  return plsc.VectorSubcoreMesh(core_axis_name="c", subcore_axis_name="s")


_SC_PARAMS = pltpu.CompilerParams(use_tc_tiling_on_sc=False)


def _zero_fill(zbuf, width):
  @pl.loop(0, ZROWS)
  def _(r):
    for k in range(width // 16):
      zbuf[r, pl.ds(k * 16, 16)] = jnp.zeros((16,), jnp.float32)


def _zero_acc(acc, zbuf, t0):
  for j in range(STRIPE // ZROWS):
    pltpu.sync_copy(zbuf, acc.at[pl.ds(t0 + j * ZROWS, ZROWS)])


def _deg_body(dst_hbm, out_hbm, acc, dstidx, ones, zbuf, sem):
  c = lax.axis_index("c")
  s = lax.axis_index("s")
  w = s * 2 + c
  t0 = s * STRIPE
  _zero_fill(zbuf, 16)

  @pl.loop(0, 256)
  def _(r):
    ones[r, :] = jnp.ones((16,), jnp.float32)

  _zero_acc(acc, zbuf, t0)
  plsc.subcore_barrier()

  @pl.loop(0, 98)
  def _(i):
    pltpu.sync_copy(dst_hbm.at[w, i], dstidx)
    pltpu.sync_copy(ones, acc.at[dstidx], add=True)

  plsc.subcore_barrier()
  pltpu.sync_copy(acc.at[pl.ds(t0, STRIPE)], out_hbm.at[c, pl.ds(t0, STRIPE)])


def _sc_deg(dst_p):
  return pl.kernel(
      _deg_body,
      out_type=jax.ShapeDtypeStruct((2, NROW, 16), jnp.float32),
      mesh=_sc_mesh(),
      compiler_params=_SC_PARAMS,
      scratch_types=[
          pltpu.VMEM_SHARED((NROW, 16), jnp.float32),
          pltpu.VMEM((256,), jnp.int32),
          pltpu.VMEM((256, 16), jnp.float32),
          pltpu.VMEM((ZROWS, 16), jnp.float32),
          pltpu.SemaphoreType.DMA,
      ],
  )(dst_p)


def _make_agg_body(n_slices, width, bsz, nslot, split):
  shares = 16 if split else NTILE
  eps = EPAD // shares
  nb = eps // bsz
  no = nb // nslot

  def body(src_hbm, dst_hbm, *refs):
    gts = refs[:n_slices]
    out_hbm = refs[n_slices]
    acc, zbuf = refs[n_slices + 1:n_slices + 3]
    base = n_slices + 3
    srcb = refs[base:base + nslot]
    dstb = refs[base + nslot:base + 2 * nslot]
    rows = refs[base + 2 * nslot:base + 3 * nslot]
    isems = refs[base + 3 * nslot:base + 4 * nslot]
    gsems = refs[base + 4 * nslot:base + 5 * nslot]
    ssems = refs[base + 5 * nslot:base + 6 * nslot]
    c = lax.axis_index("c")
    s = lax.axis_index("s")
    share = s if split else s * 2 + c
    t0 = s * STRIPE
    _zero_fill(zbuf, width)

    def sidx(g, j):
      return pltpu.make_async_copy(src_hbm.at[share, pl.ds(g * bsz, bsz)],
                                   srcb[j], isems[j])

    def didx(g, j):
      return pltpu.make_async_copy(dst_hbm.at[share, g], dstb[j], isems[j])

    for sl in range(n_slices):
      table = gts[sl]

      def gth(j, table=table):
        return pltpu.make_async_copy(table.at[srcb[j]], rows[j], gsems[j])

      def sct(j):
        return pltpu.make_async_copy(rows[j], acc.at[dstb[j]], ssems[j])

      def one_pass():
        _zero_acc(acc, zbuf, t0)
        plsc.subcore_barrier()
        for j in range(nslot):
          sidx(j, j).start()
          didx(j, j).start()
        for j in range(nslot):
          sidx(j, j).wait()
          didx(j, j).wait()
          gth(j).start()

        @pl.loop(0, no)
        def _(o):
          g0 = o * nslot
          for j in range(nslot):
            g = g0 + j
            gth(j).wait()

            @pl.when(g < nb - nslot)
            def _():
              sidx(g + nslot, j).start()

            sct(j).start(add=True)
          for j in range(nslot):
            g = g0 + j
            sct(j).wait()

            @pl.when(g < nb - nslot)
            def _():
              didx(g + nslot, j).start()
              sidx(g + nslot, j).wait()
              didx(g + nslot, j).wait()
              gth(j).start()

        plsc.subcore_barrier()
        out_k = sl if split else 2 * sl + c
        pltpu.sync_copy(acc.at[pl.ds(t0, STRIPE)],
                        out_hbm.at[out_k, pl.ds(t0, STRIPE)])
        plsc.subcore_barrier()

      if split:
        @pl.when(c == sl // (n_slices // 2))
        def _():
          one_pass()
      else:
        one_pass()

  return body


def _sc_agg(src_p, dst_p, tables, width, bsz, nslot, split):
  n_slices = len(tables)
  shares = 16 if split else NTILE
  eps = EPAD // shares
  n_out = n_slices if split else 2 * n_slices
  return pl.kernel(
      _make_agg_body(n_slices, width, bsz, nslot, split),
      out_type=jax.ShapeDtypeStruct((n_out, NROW, width), jnp.float32),
      mesh=_sc_mesh(),
      compiler_params=_SC_PARAMS,
      scratch_types=[
          pltpu.VMEM_SHARED((NROW, width), jnp.float32),
          pltpu.VMEM((ZROWS, width), jnp.float32),
      ] + [pltpu.VMEM((bsz,), jnp.int32) for _ in range(nslot)]
        + [pltpu.VMEM((bsz,), jnp.int32) for _ in range(nslot)]
        + [pltpu.VMEM((bsz, width), jnp.float32) for _ in range(nslot)]
        + [pltpu.SemaphoreType.DMA for _ in range(3 * nslot)],
  )(src_p.reshape(shares, eps),
    dst_p.reshape(shares, eps // bsz, bsz), *tables)


def _prep_body(dp_ref, x_ref, dinv_ref, xt_ref):
  deg = dp_ref[0] + dp_ref[1] + 1.0
  dinv = lax.rsqrt(deg)
  dinv_ref[...] = dinv
  xt_ref[...] = x_ref[...] * dinv


def _tc_prep(deg_part, x16):
  return pl.pallas_call(
      _prep_body,
      grid=(NBLK,),
      in_specs=[
          pl.BlockSpec((2, RBLK, 16), lambda i: (0, i, 0)),
          pl.BlockSpec((RBLK, 16), lambda i: (i, 0)),
      ],
      out_specs=[
          pl.BlockSpec((RBLK, 16), lambda i: (i, 0)),
          pl.BlockSpec((RBLK, 16), lambda i: (i, 0)),
      ],
      out_shape=[
          jax.ShapeDtypeStruct((NROW, 16), jnp.float32),
          jax.ShapeDtypeStruct((NROW, 16), jnp.float32),
      ],
  )(deg_part, x16)


def _mid_body(ap_ref, xt_ref, dinv_ref, w1_ref, b1_ref, w2_ref, *out_refs):
  dinv = dinv_ref[...]
  s1 = dinv * (ap_ref[0] + ap_ref[1] + xt_ref[...])
  h1 = _selu(jnp.dot(s1, w1_ref[...],
                     preferred_element_type=jnp.float32) + b1_ref[...])
  dcol = dinv[:, 0:1]
  for sl in range(SLICES):
    g = jnp.dot(h1, w2_ref[sl], preferred_element_type=jnp.float32)
    out_refs[sl][...] = g * dcol


def _tc_mid(agg1_part, xt, dinv16, W1p, b1, W2r):
  return pl.pallas_call(
      _mid_body,
      grid=(NBLK,),
      in_specs=[
          pl.BlockSpec((2, RBLK, 16), lambda i: (0, i, 0)),
          pl.BlockSpec((RBLK, 16), lambda i: (i, 0)),
          pl.BlockSpec((RBLK, 16), lambda i: (i, 0)),
          pl.BlockSpec((16, 256), lambda i: (0, 0)),
          pl.BlockSpec((1, 256), lambda i: (0, 0)),
          pl.BlockSpec((SLICES, 256, 32), lambda i: (0, 0, 0)),
      ],
      out_specs=[pl.BlockSpec((RBLK, 32), lambda i: (i, 0))
                 for _ in range(SLICES)],
      out_shape=[jax.ShapeDtypeStruct((NROW, 32), jnp.float32)
                 for _ in range(SLICES)],
  )(agg1_part, xt, dinv16, W1p, b1, W2r)


def _final_body(ap_ref, *refs):
  gt_refs = refs[:SLICES]
  (dinv_ref, bt_ref, b2_ref, fc1w_ref, fc1b_ref, fc2w_ref, fc2b_ref,
   out_ref, sums_ref, counts_ref) = refs[SLICES:]
  i = pl.program_id(0)

  @pl.when(i == 0)
  def _():
    sums_ref[...] = jnp.zeros_like(sums_ref)
    counts_ref[...] = jnp.zeros_like(counts_ref)

  dcol = dinv_ref[...][:, 0:1]
  b = bt_ref[0]
  gids = lax.broadcasted_iota(jnp.int32, (G, RBLK), 0)
  p = (gids == b).astype(jnp.float32)
  counts_ref[...] += jnp.dot(p, jnp.ones((RBLK, 128), jnp.float32),
                             preferred_element_type=jnp.float32)
  for sl in range(SLICES):
    agg = ap_ref[sl] + gt_refs[sl][...]
    h2 = _selu(dcol * agg + b2_ref[0, sl])
    sums_ref[sl] += jnp.dot(p, h2, preferred_element_type=jnp.float32)

  @pl.when(i == NBLK - 1)
  def _():
    cnt = jnp.maximum(counts_ref[...][:, 0:1], 1.0)
    z1 = jnp.zeros((G, 128), jnp.float32)
    for sl in range(SLICES):
      pooled = _selu(sums_ref[sl] / cnt)
      z1 = z1 + jnp.dot(pooled, fc1w_ref[sl],
                        preferred_element_type=jnp.float32)
    a1 = _selu(z1 + fc1b_ref[...])
    z2 = jnp.dot(a1, fc2w_ref[...],
                 preferred_element_type=jnp.float32) + fc2b_ref[...]
    m = jnp.max(z2, axis=1, keepdims=True)
    lse = m + jnp.log(jnp.sum(jnp.exp(z2 - m), axis=1, keepdims=True))
    out_ref[...] = z2 - lse


def _tc_final(agg2_part, gts, dinv16, batch3, b2r, fc1r, fc1b, fc2w, fc2b):
  return pl.pallas_call(
      _final_body,
      grid=(NBLK,),
      in_specs=[
          pl.BlockSpec((SLICES, RBLK, 32), lambda i: (0, i, 0)),
      ] + [pl.BlockSpec((RBLK, 32), lambda i: (i, 0))
           for _ in range(SLICES)] + [
          pl.BlockSpec((RBLK, 16), lambda i: (i, 0)),
          pl.BlockSpec((1, 1, RBLK), lambda i: (i, 0, 0)),
          pl.BlockSpec((1, SLICES, 32), lambda i: (0, 0, 0)),
          pl.BlockSpec((SLICES, 32, 128), lambda i: (0, 0, 0)),
          pl.BlockSpec((1, 128), lambda i: (0, 0)),
          pl.BlockSpec((128, 2), lambda i: (0, 0)),
          pl.BlockSpec((1, 2), lambda i: (0, 0)),
      ],
      out_specs=pl.BlockSpec((G, 2), lambda i: (0, 0)),
      out_shape=jax.ShapeDtypeStruct((G, 2), jnp.float32),
      scratch_shapes=[
          pltpu.VMEM((SLICES, G, 32), jnp.float32),
          pltpu.VMEM((G, 128), jnp.float32),
      ],
  )(agg2_part, *gts, dinv16, batch3, b2r, fc1r, fc1b, fc2w, fc2b)


@jax.jit
def kernel(x, edge_index, batch, W1, b1, W2, b2, fc1_w, fc1_b, fc2_w, fc2_b):
  src = edge_index[0]
  dst = edge_index[1]
  pad = EPAD - E
  src_p = jnp.concatenate([src, jnp.zeros((pad,), jnp.int32)])
  dst_p = jnp.concatenate([dst, jnp.full((pad,), GARBAGE, jnp.int32)])

  x16 = jnp.zeros((NROW, 16), jnp.float32).at[:N, :4].set(x)
  batch3 = jnp.full((NROW,), -1, jnp.int32).at[:N].set(batch)
  batch3 = batch3.reshape(NBLK, 1, RBLK)

  W1p = jnp.zeros((16, 2 * 128), jnp.float32).at[:4].set(W1)
  b1r = b1.reshape(1, 256)
  W2r = jnp.transpose(W2.reshape(256, SLICES, 32), (1, 0, 2))
  b2r = b2.reshape(1, SLICES, 32)
  fc1r = fc1_w.reshape(SLICES, 32, 128)
  fc1b = fc1_b.reshape(1, 128)
  fc2b = fc2_b.reshape(1, 2)

  deg_part = _sc_deg(dst_p.reshape(NTILE, 98, 256))
  dinv16, xt = _tc_prep(deg_part, x16)
  agg1_part = _sc_agg(src_p, dst_p, [xt], 16, 256, 7, False)
  gts = _tc_mid(agg1_part, xt, dinv16, W1p, b1r, W2r)
  agg2_part = _sc_agg(src_p, dst_p, list(gts), 32, 256, 2, True)
  return _tc_final(agg2_part, gts, dinv16, batch3, b2r, fc1r, fc1b,
                   fc2_w, fc2b)
